# baseline (speedup 1.0000x reference)
import functools

import jax
import jax.numpy as jnp
from jax import lax
from jax.experimental import pallas as pl
from jax.experimental.pallas import tpu as pltpu
from jax.experimental.pallas import tpu_sc as plsc

try:
    _info = plsc.get_sparse_core_info()
    _NC, _NS = _info.num_cores, _info.num_subcores
except Exception:
    _NC, _NS = 2, 16
_NW = _NC * _NS

_CHUNK = 128


def _build_embed(B, V, D, b_per_w, n_chunks):
    mesh = plsc.VectorSubcoreMesh(core_axis_name="c", subcore_axis_name="s")
    depth = 2

    @functools.partial(
        pl.kernel,
        mesh=mesh,
        out_type=jax.ShapeDtypeStruct((B, D), jnp.float32),
        scratch_types=[
            pltpu.VMEM((n_chunks, _CHUNK), jnp.int32),
            pltpu.VMEM((b_per_w, D), jnp.float32),
            pltpu.SemaphoreType.DMA((n_chunks,)),
            pltpu.SemaphoreType.DMA,
        ],
    )
    def _embed(table_hbm, idx_hbm, out_hbm, idx_v, rows_v, gsem, osem):
        wid = lax.axis_index("s") * _NC + lax.axis_index("c")
        with jax.named_scope("idx_load"):
            pltpu.sync_copy(idx_hbm.at[wid], idx_v)

        def gather(j):
            return pltpu.async_copy(
                table_hbm.at[idx_v.at[j]],
                rows_v.at[pl.ds(j * _CHUNK, _CHUNK)],
                gsem.at[j],
            )

        with jax.named_scope("gather_fire"):
            gathers = {j: gather(j) for j in range(depth)}
        writes = []
        for j in range(n_chunks):
            with jax.named_scope(f"gwait{j}"):
                gathers[j].wait()
            with jax.named_scope(f"wfire{j}"):
                writes.append(
                    pltpu.async_copy(
                        rows_v.at[pl.ds(j * _CHUNK, _CHUNK)],
                        out_hbm.at[pl.ds(wid * b_per_w + j * _CHUNK, _CHUNK)],
                        osem,
                    )
                )
            if j + depth < n_chunks:
                gathers[j + depth] = gather(j + depth)
        with jax.named_scope("wdrain"):
            for c in writes:
                c.wait()

    return _embed


@jax.jit
def kernel(labels, embedding):
    (B,) = labels.shape
    V, D = embedding.shape
    b_per_w = B // _NW
    n_chunks = b_per_w // _CHUNK
    idx = labels.astype(jnp.int32).reshape(_NW, n_chunks, _CHUNK)
    return _build_embed(B, V, D, b_per_w, n_chunks)(embedding, idx)


# speedup vs baseline: 1.0298x; 1.0298x over previous
import functools

import jax
import jax.numpy as jnp
from jax import lax
from jax.experimental import pallas as pl
from jax.experimental.pallas import tpu as pltpu
from jax.experimental.pallas import tpu_sc as plsc

try:
    _info = plsc.get_sparse_core_info()
    _NC, _NS = _info.num_cores, _info.num_subcores
except Exception:
    _NC, _NS = 2, 16
_NW = _NC * _NS

_CHUNK = 128


def _build_embed(B, V, D, b_per_w, n_chunks):
    mesh = plsc.VectorSubcoreMesh(core_axis_name="c", subcore_axis_name="s")

    @functools.partial(
        pl.kernel,
        mesh=mesh,
        out_type=jax.ShapeDtypeStruct((B, D), jnp.float32),
        scratch_types=[
            pltpu.VMEM((n_chunks, _CHUNK), jnp.int32),
            pltpu.VMEM((b_per_w, D), jnp.float32),
            pltpu.SemaphoreType.DMA((n_chunks,)),
            pltpu.SemaphoreType.DMA,
            pltpu.SemaphoreType.DMA,
        ],
    )
    def _embed(table_hbm, idx_hbm, out_hbm, idx_v, rows_v, gsem, osem, isem):
        wid = lax.axis_index("s") * _NC + lax.axis_index("c")

        def gather(j):
            return pltpu.async_copy(
                table_hbm.at[idx_v.at[j]],
                rows_v.at[pl.ds(j * _CHUNK, _CHUNK)],
                gsem.at[j],
            )

        i0 = pltpu.async_copy(idx_hbm.at[wid].at[pl.ds(0, 1)],
                              idx_v.at[pl.ds(0, 1)], isem)
        i1 = pltpu.async_copy(idx_hbm.at[wid].at[pl.ds(1, n_chunks - 1)],
                              idx_v.at[pl.ds(1, n_chunks - 1)], isem)
        i0.wait()
        gathers = [gather(0)]
        i1.wait()
        gathers += [gather(j) for j in range(1, n_chunks)]
        writes = []
        for j in range(n_chunks):
            gathers[j].wait()
            writes.append(
                pltpu.async_copy(
                    rows_v.at[pl.ds(j * _CHUNK, _CHUNK)],
                    out_hbm.at[pl.ds(wid * b_per_w + j * _CHUNK, _CHUNK)],
                    osem,
                )
            )
        for c in writes:
            c.wait()

    return _embed


@jax.jit
def kernel(labels, embedding):
    (B,) = labels.shape
    V, D = embedding.shape
    b_per_w = B // _NW
    n_chunks = b_per_w // _CHUNK
    idx = labels.astype(jnp.int32).reshape(_NW, n_chunks, _CHUNK)
    return _build_embed(B, V, D, b_per_w, n_chunks)(embedding, idx)
